# baseline (speedup 1.0000x reference)
import jax
import jax.numpy as jnp
from jax import lax
from jax.experimental import pallas as pl
from jax.experimental.pallas import tpu as pltpu
from jax.experimental.pallas import tpu_sc as plsc

_N, _E, _D, _DE, _DG, _H, _GH = 10000, 320000, 128, 16, 16, 32, 16
_NC, _NS = 2, 16
_NW = _NC * _NS
_EW = _E // _NW
_CH = 80
_NCHUNK = _EW // _CH
_NBUF = 4
_NPAD = 10240
_RPT = _NPAD // _NS
_EBLK = 20000

_PREC = lax.Precision.HIGHEST


def _leaky(x):
    return jnp.maximum(x, 0.01 * x)


def _node_proj_body(nodes_ref, wc_ref, ps_ref, pr_ref, nsum_ref):
    n = nodes_ref[...]
    p = jnp.dot(n, wc_ref[...], preferred_element_type=jnp.float32,
                precision=_PREC)
    ps_ref[...] = p[:, :2 * _H].astype(jnp.bfloat16)
    pr_ref[...] = p[:, 2 * _H:].astype(jnp.bfloat16)
    nsum_ref[...] = jnp.sum(n, axis=0, keepdims=True)


_node_proj = pl.pallas_call(
    _node_proj_body,
    out_shape=(jax.ShapeDtypeStruct((_N, 2 * _H), jnp.bfloat16),
               jax.ShapeDtypeStruct((_N, 2 * _H), jnp.bfloat16),
               jax.ShapeDtypeStruct((1, _D), jnp.float32)),
)


_EFOLD = 8
_ER = _E // _EFOLD
_ECOL = _EFOLD * _DE
_PCOL = _EFOLD * 2 * _H
_EBLK2 = 4000


def _edge_proj_body(g_ref, wgc_ref, bc_ref, e_ref, wblk_ref, pe_ref, esum_ref):
    i = pl.program_id(0)
    blk = e_ref[...]
    cvec = jnp.dot(g_ref[...], wgc_ref[...], preferred_element_type=jnp.float32,
                   precision=_PREC) + bc_ref[...]
    cfull = jnp.concatenate([cvec] * _EFOLD, axis=1)
    pe_ref[...] = (jnp.dot(blk, wblk_ref[...],
                           preferred_element_type=jnp.float32,
                           precision=_PREC) + cfull).astype(jnp.bfloat16)

    @pl.when(i == 0)
    def _init():
        esum_ref[...] = jnp.zeros_like(esum_ref)

    s128 = jnp.sum(blk, axis=0, keepdims=True)
    s = s128[:, :_DE]
    for j in range(1, _EFOLD):
        s = s + s128[:, j * _DE:(j + 1) * _DE]
    esum_ref[...] = esum_ref[...] + s


_edge_proj = pl.pallas_call(
    _edge_proj_body,
    grid=(_ER // _EBLK2,),
    in_specs=[pl.BlockSpec((1, _DG), lambda i: (0, 0)),
              pl.BlockSpec((_DG, 2 * _H), lambda i: (0, 0)),
              pl.BlockSpec((1, 2 * _H), lambda i: (0, 0)),
              pl.BlockSpec((_EBLK2, _ECOL), lambda i: (i, 0)),
              pl.BlockSpec((_ECOL, _PCOL), lambda i: (0, 0))],
    out_specs=(pl.BlockSpec((_EBLK2, _PCOL), lambda i: (i, 0)),
               pl.BlockSpec((1, _DE), lambda i: (0, 0))),
    out_shape=(jax.ShapeDtypeStruct((_ER, _PCOL), jnp.bfloat16),
               jax.ShapeDtypeStruct((1, _DE), jnp.float32)),
)


def _global_body(nsum_ref, esum_ref, g_ref, wgn_ref, bgn_ref, wge_ref,
                 bge_ref, wg_ref, bg_ref, wfg_ref, bfg_ref, out_ref):
    tn = _leaky(jnp.dot(nsum_ref[...], wgn_ref[...],
                        preferred_element_type=jnp.float32,
                        precision=_PREC) + bgn_ref[...])
    te = _leaky(jnp.dot(esum_ref[...], wge_ref[...],
                        preferred_element_type=jnp.float32,
                        precision=_PREC) + bge_ref[...])
    tg = _leaky(jnp.dot(g_ref[...], wg_ref[...],
                        preferred_element_type=jnp.float32,
                        precision=_PREC) + bg_ref[...])
    fa = jnp.concatenate([tg, tn, te], axis=1)
    out_ref[...] = _leaky(jnp.dot(fa, wfg_ref[...],
                                  preferred_element_type=jnp.float32,
                                  precision=_PREC) + bfg_ref[...])


_global_update = pl.pallas_call(
    _global_body,
    out_shape=jax.ShapeDtypeStruct((1, _GH), jnp.float32),
)


_PER = _CH // _EFOLD
_PEW = _EW // _EFOLD


def _sc_body(ps_hbm, pr_hbm, pe_hbm, sidx_hbm, ridx_hbm, zeros_hbm,
             ne_hbm, part_hbm,
             sidx_v, ridx_v, *rest):
    bufs = []
    for b in range(_NBUF):
        bufs.append(tuple(rest[b * 5:(b + 1) * 5])
                    + tuple(rest[_NBUF * 5 + 1 + b * 5:
                                 _NBUF * 5 + 1 + (b + 1) * 5]))
    acc_sh = rest[_NBUF * 5]
    c_ax = lax.axis_index("c")
    s_ax = lax.axis_index("s")
    wid = c_ax * _NS + s_ax
    pltpu.sync_copy(zeros_hbm.at[pl.ds(s_ax * _RPT, _RPT)],
                    acc_sh.at[pl.ds(s_ax * _RPT, _RPT)])
    pltpu.sync_copy(sidx_hbm.at[pl.ds(wid * _EW, _EW)], sidx_v)
    pltpu.sync_copy(ridx_hbm.at[wid], ridx_v)
    plsc.subcore_barrier()
    ebase = wid * _EW
    pebase = wid * _PEW

    def start_gathers(c, bb):
        ps_t, pr_t, pe_t = bb[0], bb[1], bb[2]
        s1, s2, s3 = bb[5], bb[6], bb[7]
        pltpu.async_copy(ps_hbm.at[sidx_v.at[pl.ds(c * _CH, _CH)]], ps_t, s1)
        pltpu.async_copy(pr_hbm.at[ridx_v.at[c]], pr_t, s2)
        pltpu.async_copy(pe_hbm.at[pl.ds(pebase + c * _PER, _PER)], pe_t, s3)

    def wait_gathers(c, bb):
        ps_t, pr_t, pe_t = bb[0], bb[1], bb[2]
        s1, s2, s3 = bb[5], bb[6], bb[7]
        pltpu.make_async_copy(
            ps_hbm.at[sidx_v.at[pl.ds(c * _CH, _CH)]], ps_t, s1).wait()
        pltpu.make_async_copy(pr_hbm.at[ridx_v.at[c]], pr_t, s2).wait()
        pltpu.make_async_copy(
            pe_hbm.at[pl.ds(pebase + c * _PER, _PER)], pe_t, s3).wait()

    def wait_stores(c, bb):
        msg_t, eout_t = bb[3], bb[4]
        s4, s5 = bb[8], bb[9]
        pltpu.make_async_copy(
            eout_t, ne_hbm.at[pl.ds(ebase + c * _CH, _CH)], s4).wait()
        pltpu.make_async_copy(msg_t, acc_sh.at[ridx_v.at[c]], s5).wait()

    def compute(bb):
        ps_t, pr_t, pe_t, msg_t, eout_t = bb[:5]

        def row_body(er, carry):
            for sub in range(8):
                e = er * 8 + sub
                for h in range(2):
                    sl = pl.ds(32 * h, 32)
                    pss = plsc.unpack(ps_t[e, sl],
                                      format=plsc.PackFormat.INTERLEAVED)
                    prs = plsc.unpack(pr_t[e, sl],
                                      format=plsc.PackFormat.INTERLEAVED)
                    pes = plsc.unpack(pe_t[er, pl.ds(sub * 64 + 32 * h, 32)],
                                      format=plsc.PackFormat.INTERLEAVED)
                    for q in range(2):
                        g = 2 * h + q
                        v = pss[q] + prs[q] + pes[q]
                        o = jnp.maximum(v, 0.01 * v)
                        if g < 2:
                            msg_t[e, pl.ds(16 * g, 16)] = o
                        else:
                            eout_t[e, pl.ds(16 * (g - 2), 16)] = o
            return carry

        lax.fori_loop(0, _CH // 8, row_body, 0)

    def chunk_step(c, b, traced):
        bb = bufs[b]
        msg_t, eout_t = bb[3], bb[4]
        s4, s5 = bb[8], bb[9]
        wait_gathers(c, bb)
        nxt = bufs[(b + _NBUF - 1) % _NBUF]
        if traced:
            pl.when(c + _NBUF - 1 < _NCHUNK)(
                lambda: start_gathers(c + _NBUF - 1, nxt))
            pl.when(c >= _NBUF)(lambda: wait_stores(c - _NBUF, bb))
        else:
            if c + _NBUF - 1 < _NCHUNK:
                start_gathers(c + _NBUF - 1, nxt)
            if c >= _NBUF:
                wait_stores(c - _NBUF, bb)
        compute(bb)
        pltpu.async_copy(eout_t, ne_hbm.at[pl.ds(ebase + c * _CH, _CH)], s4)
        pltpu.async_copy(msg_t, acc_sh.at[ridx_v.at[c]], s5, add=True)

    for b in range(_NBUF - 1):
        start_gathers(b, bufs[b])

    def quad(j, carry):
        c0 = j * _NBUF
        for b in range(_NBUF):
            chunk_step(c0 + b, b, True)
        return carry

    lax.fori_loop(0, _NCHUNK // _NBUF, quad, 0)
    chunk_step(_NCHUNK - 1, 0, False)
    for k in range(_NBUF):
        c = _NCHUNK - _NBUF + k
        wait_stores(c, bufs[c % _NBUF])
    plsc.subcore_barrier()
    pltpu.sync_copy(acc_sh.at[pl.ds(s_ax * _RPT, _RPT)],
                    part_hbm.at[c_ax, pl.ds(s_ax * _RPT, _RPT)])


_sc_scratch = [pltpu.VMEM((_EW,), jnp.int32),
               pltpu.VMEM((_NCHUNK, _CH), jnp.int32)]
for _b in range(_NBUF):
    _sc_scratch += [pltpu.VMEM((_CH, 4 * 16), jnp.bfloat16),
                    pltpu.VMEM((_CH, 4 * 16), jnp.bfloat16),
                    pltpu.VMEM((_PER, _PCOL), jnp.bfloat16),
                    pltpu.VMEM((_CH, _H), jnp.float32),
                    pltpu.VMEM((_CH, _H), jnp.float32)]
_sc_scratch += [pltpu.VMEM_SHARED((_NPAD, _H), jnp.float32)]
_sc_scratch += [pltpu.SemaphoreType.DMA] * (5 * _NBUF)

_sc_edges = pl.kernel(
    _sc_body,
    out_type=(jax.ShapeDtypeStruct((_E, _H), jnp.float32),
              jax.ShapeDtypeStruct((_NC, _NPAD, _H), jnp.float32)),
    mesh=plsc.VectorSubcoreMesh(core_axis_name="c", subcore_axis_name="s"),
    compiler_params=pltpu.CompilerParams(use_tc_tiling_on_sc=False,
                                        needs_layout_passes=False),
    scratch_types=_sc_scratch,
)


def _combine_body(part_ref, out_ref):
    out_ref[...] = part_ref[0, :_N, :] + part_ref[1, :_N, :]


_combine = pl.pallas_call(
    _combine_body,
    out_shape=jax.ShapeDtypeStruct((_N, _H), jnp.float32),
)


def kernel(nodes, edges, senders, receivers, globals_, n_node, n_edge,
           W_node, b_node, W_edge, b_edge, W_gn, b_gn, W_ge, b_ge,
           W_g, b_g, W_fg, b_fg):
    senders = senders.astype(jnp.int32)
    receivers = receivers.astype(jnp.int32)
    wc = jnp.concatenate(
        [jnp.concatenate([W_node[:_D], W_edge[:_D]], axis=1),
         jnp.concatenate([W_node[_D:2 * _D], W_edge[_D:2 * _D]], axis=1)],
        axis=1)
    wec = jnp.concatenate([W_node[2 * _D:2 * _D + _DE],
                           W_edge[2 * _D:2 * _D + _DE]], axis=1)
    wgc = jnp.concatenate([W_node[2 * _D + _DE:],
                           W_edge[2 * _D + _DE:]], axis=1)
    bc = jnp.concatenate([b_node, b_edge]).reshape(1, 2 * _H)

    def _ilv(w):
        r = w.shape[0]
        return w.reshape(r, 2, 2, 16).swapaxes(2, 3).reshape(r, 64)

    wc = jnp.concatenate([_ilv(wc[:, :64]), _ilv(wc[:, 64:])], axis=1)
    wec = _ilv(wec)
    wgc = _ilv(wgc)
    bc = _ilv(bc)

    wblk = jnp.kron(jnp.eye(_EFOLD, dtype=jnp.float32), wec)
    ps, pr, nsum = _node_proj(nodes, wc)
    pe, esum = _edge_proj(globals_, wgc, bc, edges.reshape(_ER, _ECOL), wblk)
    new_global = _global_update(nsum, esum, globals_,
                                W_gn, b_gn.reshape(1, -1),
                                W_ge, b_ge.reshape(1, -1),
                                W_g, b_g.reshape(1, -1),
                                W_fg, b_fg.reshape(1, -1))

    ridx2 = receivers.reshape(_NW, _NCHUNK, _CH)
    zeros = jnp.zeros((_NPAD, _H), jnp.float32)
    new_edges, part = _sc_edges(ps, pr, pe, senders, ridx2, zeros)
    new_nodes = _combine(part)
    return new_nodes, new_edges, new_global


# speedup vs baseline: 1.0888x; 1.0888x over previous
import jax
import jax.numpy as jnp
from jax import lax
from jax.experimental import pallas as pl
from jax.experimental.pallas import tpu as pltpu
from jax.experimental.pallas import tpu_sc as plsc

_N, _E, _D, _DE, _DG, _H, _GH = 10000, 320000, 128, 16, 16, 32, 16
_NC, _NS = 2, 16
_NW = _NC * _NS
_EW = _E // _NW
_CH = 80
_NCHUNK = _EW // _CH
_NBUF = 4
_NPAD = 10240
_RPT = _NPAD // _NS
_EBLK = 20000

_PREC = lax.Precision.HIGHEST


def _leaky(x):
    return jnp.maximum(x, 0.01 * x)


def _node_proj_body(nodes_ref, wc_ref, ps_ref, pr_ref, nsum_ref):
    n = nodes_ref[...]
    p = jnp.dot(n, wc_ref[...], preferred_element_type=jnp.float32,
                precision=_PREC)
    ps_ref[...] = p[:, :2 * _H].astype(jnp.bfloat16)
    pr_ref[...] = p[:, 2 * _H:].astype(jnp.bfloat16)
    nsum_ref[...] = jnp.sum(n, axis=0, keepdims=True)


_node_proj = pl.pallas_call(
    _node_proj_body,
    out_shape=(jax.ShapeDtypeStruct((_N, 2 * _H), jnp.bfloat16),
               jax.ShapeDtypeStruct((_N, 2 * _H), jnp.bfloat16),
               jax.ShapeDtypeStruct((1, _D), jnp.float32)),
)


_EFOLD = 8
_ER = _E // _EFOLD
_ECOL = _EFOLD * _DE
_PCOL = _EFOLD * 2 * _H
_EBLK2 = 4000


def _edge_proj_body(g_ref, wgc_ref, bc_ref, e_ref, wblk_ref, pe_ref, esum_ref):
    i = pl.program_id(0)
    blk = e_ref[...]
    cvec = jnp.dot(g_ref[...], wgc_ref[...], preferred_element_type=jnp.float32,
                   precision=_PREC) + bc_ref[...]
    cfull = jnp.concatenate([cvec] * _EFOLD, axis=1)
    pe_ref[...] = (jnp.dot(blk, wblk_ref[...],
                           preferred_element_type=jnp.float32,
                           precision=lax.Precision.DEFAULT)
                   + cfull).astype(jnp.bfloat16)

    @pl.when(i == 0)
    def _init():
        esum_ref[...] = jnp.zeros_like(esum_ref)

    s128 = jnp.sum(blk, axis=0, keepdims=True)
    s = s128[:, :_DE]
    for j in range(1, _EFOLD):
        s = s + s128[:, j * _DE:(j + 1) * _DE]
    esum_ref[...] = esum_ref[...] + s


_edge_proj = pl.pallas_call(
    _edge_proj_body,
    grid=(_ER // _EBLK2,),
    in_specs=[pl.BlockSpec((1, _DG), lambda i: (0, 0)),
              pl.BlockSpec((_DG, 2 * _H), lambda i: (0, 0)),
              pl.BlockSpec((1, 2 * _H), lambda i: (0, 0)),
              pl.BlockSpec((_EBLK2, _ECOL), lambda i: (i, 0)),
              pl.BlockSpec((_ECOL, _PCOL), lambda i: (0, 0))],
    out_specs=(pl.BlockSpec((_EBLK2, _PCOL), lambda i: (i, 0)),
               pl.BlockSpec((1, _DE), lambda i: (0, 0))),
    out_shape=(jax.ShapeDtypeStruct((_ER, _PCOL), jnp.bfloat16),
               jax.ShapeDtypeStruct((1, _DE), jnp.float32)),
)


def _global_body(nsum_ref, esum_ref, g_ref, wgn_ref, bgn_ref, wge_ref,
                 bge_ref, wg_ref, bg_ref, wfg_ref, bfg_ref, out_ref):
    tn = _leaky(jnp.dot(nsum_ref[...], wgn_ref[...],
                        preferred_element_type=jnp.float32,
                        precision=_PREC) + bgn_ref[...])
    te = _leaky(jnp.dot(esum_ref[...], wge_ref[...],
                        preferred_element_type=jnp.float32,
                        precision=_PREC) + bge_ref[...])
    tg = _leaky(jnp.dot(g_ref[...], wg_ref[...],
                        preferred_element_type=jnp.float32,
                        precision=_PREC) + bg_ref[...])
    fa = jnp.concatenate([tg, tn, te], axis=1)
    out_ref[...] = _leaky(jnp.dot(fa, wfg_ref[...],
                                  preferred_element_type=jnp.float32,
                                  precision=_PREC) + bfg_ref[...])


_global_update = pl.pallas_call(
    _global_body,
    out_shape=jax.ShapeDtypeStruct((1, _GH), jnp.float32),
)


_PER = _CH // _EFOLD
_PEW = _EW // _EFOLD


def _sc_body(ps_hbm, pr_hbm, pe_hbm, sidx_hbm, ridx_hbm, zeros_hbm,
             ne_hbm, part_hbm,
             sidx_v, ridx_v, *rest):
    bufs = []
    for b in range(_NBUF):
        bufs.append(tuple(rest[b * 5:(b + 1) * 5])
                    + tuple(rest[_NBUF * 5 + 1 + b * 5:
                                 _NBUF * 5 + 1 + (b + 1) * 5]))
    acc_sh = rest[_NBUF * 5]
    c_ax = lax.axis_index("c")
    s_ax = lax.axis_index("s")
    wid = c_ax * _NS + s_ax
    pltpu.sync_copy(zeros_hbm.at[pl.ds(s_ax * _RPT, _RPT)],
                    acc_sh.at[pl.ds(s_ax * _RPT, _RPT)])
    pltpu.sync_copy(sidx_hbm.at[pl.ds(wid * _EW, _EW)], sidx_v)
    pltpu.sync_copy(ridx_hbm.at[wid], ridx_v)
    plsc.subcore_barrier()
    ebase = wid * _EW
    pebase = wid * _PEW

    def start_gathers(c, bb):
        ps_t, pr_t, pe_t = bb[0], bb[1], bb[2]
        s1, s2, s3 = bb[5], bb[6], bb[7]
        pltpu.async_copy(ps_hbm.at[sidx_v.at[pl.ds(c * _CH, _CH)]], ps_t, s1)
        pltpu.async_copy(pr_hbm.at[ridx_v.at[c]], pr_t, s2)
        pltpu.async_copy(pe_hbm.at[pl.ds(pebase + c * _PER, _PER)], pe_t, s3)

    def wait_gathers(c, bb):
        ps_t, pr_t, pe_t = bb[0], bb[1], bb[2]
        s1, s2, s3 = bb[5], bb[6], bb[7]
        pltpu.make_async_copy(
            ps_hbm.at[sidx_v.at[pl.ds(c * _CH, _CH)]], ps_t, s1).wait()
        pltpu.make_async_copy(pr_hbm.at[ridx_v.at[c]], pr_t, s2).wait()
        pltpu.make_async_copy(
            pe_hbm.at[pl.ds(pebase + c * _PER, _PER)], pe_t, s3).wait()

    def wait_stores(c, bb):
        msg_t, eout_t = bb[3], bb[4]
        s4, s5 = bb[8], bb[9]
        pltpu.make_async_copy(
            eout_t, ne_hbm.at[pl.ds(ebase + c * _CH, _CH)], s4).wait()
        pltpu.make_async_copy(msg_t, acc_sh.at[ridx_v.at[c]], s5).wait()

    def compute(bb):
        ps_t, pr_t, pe_t, msg_t, eout_t = bb[:5]

        def row_body(er, carry):
            for sub in range(8):
                e = er * 8 + sub
                for h in range(2):
                    sl = pl.ds(32 * h, 32)
                    pss = plsc.unpack(ps_t[e, sl],
                                      format=plsc.PackFormat.INTERLEAVED)
                    prs = plsc.unpack(pr_t[e, sl],
                                      format=plsc.PackFormat.INTERLEAVED)
                    pes = plsc.unpack(pe_t[er, pl.ds(sub * 64 + 32 * h, 32)],
                                      format=plsc.PackFormat.INTERLEAVED)
                    for q in range(2):
                        g = 2 * h + q
                        v = pss[q] + prs[q] + pes[q]
                        o = jnp.maximum(v, 0.01 * v)
                        if g < 2:
                            msg_t[e, pl.ds(16 * g, 16)] = o
                        else:
                            eout_t[e, pl.ds(16 * (g - 2), 16)] = o
            return carry

        lax.fori_loop(0, _CH // 8, row_body, 0)

    def chunk_step(c, b, traced):
        bb = bufs[b]
        msg_t, eout_t = bb[3], bb[4]
        s4, s5 = bb[8], bb[9]
        wait_gathers(c, bb)
        nxt = bufs[(b + _NBUF - 1) % _NBUF]
        if traced:
            pl.when(c + _NBUF - 1 < _NCHUNK)(
                lambda: start_gathers(c + _NBUF - 1, nxt))
            pl.when(c >= _NBUF)(lambda: wait_stores(c - _NBUF, bb))
        else:
            if c + _NBUF - 1 < _NCHUNK:
                start_gathers(c + _NBUF - 1, nxt)
            if c >= _NBUF:
                wait_stores(c - _NBUF, bb)
        compute(bb)
        pltpu.async_copy(eout_t, ne_hbm.at[pl.ds(ebase + c * _CH, _CH)], s4)
        pltpu.async_copy(msg_t, acc_sh.at[ridx_v.at[c]], s5, add=True)

    for b in range(_NBUF - 1):
        start_gathers(b, bufs[b])

    def quad(j, carry):
        c0 = j * _NBUF
        for b in range(_NBUF):
            chunk_step(c0 + b, b, True)
        return carry

    lax.fori_loop(0, _NCHUNK // _NBUF, quad, 0)
    chunk_step(_NCHUNK - 1, 0, False)
    for k in range(_NBUF):
        c = _NCHUNK - _NBUF + k
        wait_stores(c, bufs[c % _NBUF])
    plsc.subcore_barrier()
    pltpu.sync_copy(acc_sh.at[pl.ds(s_ax * _RPT, _RPT)],
                    part_hbm.at[c_ax, pl.ds(s_ax * _RPT, _RPT)])


_sc_scratch = [pltpu.VMEM((_EW,), jnp.int32),
               pltpu.VMEM((_NCHUNK, _CH), jnp.int32)]
for _b in range(_NBUF):
    _sc_scratch += [pltpu.VMEM((_CH, 4 * 16), jnp.bfloat16),
                    pltpu.VMEM((_CH, 4 * 16), jnp.bfloat16),
                    pltpu.VMEM((_PER, _PCOL), jnp.bfloat16),
                    pltpu.VMEM((_CH, _H), jnp.float32),
                    pltpu.VMEM((_CH, _H), jnp.float32)]
_sc_scratch += [pltpu.VMEM_SHARED((_NPAD, _H), jnp.float32)]
_sc_scratch += [pltpu.SemaphoreType.DMA] * (5 * _NBUF)

_sc_edges = pl.kernel(
    _sc_body,
    out_type=(jax.ShapeDtypeStruct((_E, _H), jnp.float32),
              jax.ShapeDtypeStruct((_NC, _NPAD, _H), jnp.float32)),
    mesh=plsc.VectorSubcoreMesh(core_axis_name="c", subcore_axis_name="s"),
    compiler_params=pltpu.CompilerParams(use_tc_tiling_on_sc=False,
                                        needs_layout_passes=False),
    scratch_types=_sc_scratch,
)


def _combine_body(part_ref, out_ref):
    out_ref[...] = part_ref[0, :_N, :] + part_ref[1, :_N, :]


_combine = pl.pallas_call(
    _combine_body,
    out_shape=jax.ShapeDtypeStruct((_N, _H), jnp.float32),
)


def kernel(nodes, edges, senders, receivers, globals_, n_node, n_edge,
           W_node, b_node, W_edge, b_edge, W_gn, b_gn, W_ge, b_ge,
           W_g, b_g, W_fg, b_fg):
    senders = senders.astype(jnp.int32)
    receivers = receivers.astype(jnp.int32)
    wc = jnp.concatenate(
        [jnp.concatenate([W_node[:_D], W_edge[:_D]], axis=1),
         jnp.concatenate([W_node[_D:2 * _D], W_edge[_D:2 * _D]], axis=1)],
        axis=1)
    wec = jnp.concatenate([W_node[2 * _D:2 * _D + _DE],
                           W_edge[2 * _D:2 * _D + _DE]], axis=1)
    wgc = jnp.concatenate([W_node[2 * _D + _DE:],
                           W_edge[2 * _D + _DE:]], axis=1)
    bc = jnp.concatenate([b_node, b_edge]).reshape(1, 2 * _H)

    def _ilv(w):
        r = w.shape[0]
        return w.reshape(r, 2, 2, 16).swapaxes(2, 3).reshape(r, 64)

    wc = jnp.concatenate([_ilv(wc[:, :64]), _ilv(wc[:, 64:])], axis=1)
    wec = _ilv(wec)
    wgc = _ilv(wgc)
    bc = _ilv(bc)

    wblk = jnp.kron(jnp.eye(_EFOLD, dtype=jnp.float32), wec)
    ps, pr, nsum = _node_proj(nodes, wc)
    pe, esum = _edge_proj(globals_, wgc, bc, edges.reshape(_ER, _ECOL), wblk)
    new_global = _global_update(nsum, esum, globals_,
                                W_gn, b_gn.reshape(1, -1),
                                W_ge, b_ge.reshape(1, -1),
                                W_g, b_g.reshape(1, -1),
                                W_fg, b_fg.reshape(1, -1))

    ridx2 = receivers.reshape(_NW, _NCHUNK, _CH)
    zeros = jnp.zeros((_NPAD, _H), jnp.float32)
    new_edges, part = _sc_edges(ps, pr, pe, senders, ridx2, zeros)
    new_nodes = _combine(part)
    return new_nodes, new_edges, new_global
